# baseline (speedup 1.0000x reference)
import functools

import jax
import jax.numpy as jnp
from jax import lax
from jax.experimental import pallas as pl
from jax.experimental.pallas import tpu as pltpu
from jax.experimental.pallas import tpu_sc as plsc

N = 10000
D_IN = 128
D_OUT = 64
E = 320000

NC = 2
NS = 16
NW = NC * NS

EC = 128
NB = 80
NBE = NB * EC
EPAD = NW * NBE
NPAD = 10240
ROWS_PER_TILE = NPAD // NS
BLK = ROWS_PER_TILE // EC
NDUMMY = NPAD - N
NBUF = 3
VL = 16


def _prep_body(x_ref, w_ref, dst_ref, msk_ref, y_ref, dstm_ref):
    y_ref[...] = lax.dot_general(
        x_ref[...], w_ref[...],
        (((1,), (1,)), ((), ())),
        preferred_element_type=jnp.float32,
    )
    r = lax.broadcasted_iota(jnp.int32, (EPAD // EC, EC), 0)
    cc = lax.broadcasted_iota(jnp.int32, (EPAD // EC, EC), 1)
    dummy = N + lax.rem(r * EC + cc, NDUMMY)
    dstm_ref[...] = jnp.where(msk_ref[...] != 0, dst_ref[...], dummy)


def _prep(x_pad, w, dst_f, msk_f):
    return pl.pallas_call(
        _prep_body,
        out_shape=(
            jax.ShapeDtypeStruct((NPAD, D_OUT), jnp.float32),
            jax.ShapeDtypeStruct((EPAD // EC, EC), jnp.int32),
        ),
    )(x_pad, w, dst_f, msk_f)


_MESH = plsc.VectorSubcoreMesh(
    core_axis_name="c", subcore_axis_name="s", num_cores=NC, num_subcores=NS
)
_CPARAMS = pltpu.CompilerParams(use_tc_tiling_on_sc=False)

_SC_SCRATCH = [
    pltpu.VMEM((NBE,), jnp.int32),
    pltpu.VMEM((NB, EC), jnp.int32),
    pltpu.VMEM((EC, D_OUT), jnp.float32),
    pltpu.VMEM((EC, D_OUT), jnp.float32),
    pltpu.VMEM((EC, D_OUT), jnp.float32),
    pltpu.VMEM((BLK, EC), jnp.int32),
    pltpu.VMEM_SHARED((NPAD, D_OUT), jnp.float32),
    pltpu.VMEM_SHARED((NPAD, D_OUT), jnp.float32),
    pltpu.SemaphoreType.DMA,
    pltpu.SemaphoreType.DMA,
    pltpu.SemaphoreType.DMA,
    pltpu.SemaphoreType.DMA,
    pltpu.SemaphoreType.DMA,
    pltpu.SemaphoreType.DMA,
]


def _zero_buf(buf):
    z = jnp.zeros((VL,), jnp.float32)

    def zrow(i, carry):
        for f in range(D_OUT // VL):
            buf[i, pl.ds(f * VL, VL)] = z
        return carry

    lax.fori_loop(0, EC, zrow, jnp.int32(0))


def _zero_accum(accum, r0, row0):
    for blk in range(BLK):
        pltpu.sync_copy(r0, accum.at[pl.ds(row0 + blk * EC, EC)])


def _fill_rowidx(idx2, row0):
    lanes = lax.broadcasted_iota(jnp.int32, (VL,), 0)
    for blk in range(BLK):
        for k in range(EC // VL):
            idx2[blk, pl.ds(k * VL, VL)] = row0 + blk * EC + k * VL + lanes


def _ring_loop(ysp, accum, src_v, dst_v, rows, gsems, ssems):
    def issue_gather(j, b):
        pltpu.async_copy(ysp.at[src_v.at[pl.ds(j * EC, EC)]], rows[b],
                         gsems[b])

    def wait(sem):
        pltpu.make_async_copy(ysp.at[pl.ds(0, EC)], rows[0], sem).wait()

    for b in range(NBUF):
        issue_gather(b, b)

    def gbody(gi, carry):
        g = gi * NBUF
        for b in range(NBUF):
            j = g + b
            bprev = (b - 1) % NBUF
            @pl.when((j >= 1) & (j - 1 + NBUF < NB))
            def _(j=j, bprev=bprev):
                wait(ssems[bprev])
                issue_gather(j - 1 + NBUF, bprev)

            @pl.when(j < NB)
            def _(j=j, b=b):
                wait(gsems[b])
                pltpu.async_copy(rows[b], accum.at[dst_v.at[j]], ssems[b],
                                 add=True)
        return carry

    lax.fori_loop(0, (NB + NBUF - 1) // NBUF, gbody, jnp.int32(0))
    for b in range(NBUF):
        wait(ssems[b])


@functools.partial(
    pl.kernel,
    out_type=jax.ShapeDtypeStruct((NC, NS, ROWS_PER_TILE, D_OUT), jnp.float32),
    mesh=_MESH,
    scratch_types=_SC_SCRATCH,
    compiler_params=_CPARAMS,
)
def _propagate1(y_hbm, src_hbm, dst_hbm, out_hbm,
                src_v, dst_v, r0, r1, r2, idx2, ysp, accum,
                g0, g1, g2, s0, s1, s2):
    c = lax.axis_index("c")
    s = lax.axis_index("s")
    wid = c * NS + s
    row0 = s * ROWS_PER_TILE
    pltpu.sync_copy(y_hbm.at[pl.ds(row0, ROWS_PER_TILE)],
                    ysp.at[pl.ds(row0, ROWS_PER_TILE)])
    _zero_buf(r0)
    _zero_accum(accum, r0, row0)
    pltpu.sync_copy(src_hbm.at[wid], src_v)
    pltpu.sync_copy(dst_hbm.at[wid], dst_v)
    plsc.subcore_barrier()
    @pl.when(s % 2 == 0)
    def _():
        _ring_loop(ysp, accum, src_v, dst_v, [r0, r1, r2], [g0, g1, g2],
                   [s0, s1, s2])

    @pl.when(s % 2 == 1)
    def _():
        _ring_loop(y_hbm, accum, src_v, dst_v, [r0, r1, r2], [g0, g1, g2],
                   [s0, s1, s2])
    plsc.subcore_barrier()
    pltpu.sync_copy(accum.at[pl.ds(row0, ROWS_PER_TILE)], out_hbm.at[c, s])


@functools.partial(
    pl.kernel,
    out_type=jax.ShapeDtypeStruct((NC, NS, ROWS_PER_TILE, D_OUT), jnp.float32),
    mesh=_MESH,
    scratch_types=_SC_SCRATCH,
    compiler_params=_CPARAMS,
)
def _propagate2(part_hbm, src_hbm, dst_hbm, out_hbm,
                src_v, dst_v, r0, r1, r2, idx2, ysp, accum,
                g0, g1, g2, s0, s1, s2):
    c = lax.axis_index("c")
    s = lax.axis_index("s")
    wid = c * NS + s
    row0 = s * ROWS_PER_TILE
    _fill_rowidx(idx2, row0)
    for blk in range(BLK):
        pltpu.sync_copy(part_hbm.at[0, s, pl.ds(blk * EC, EC)], r0)
        pltpu.sync_copy(r0, ysp.at[pl.ds(row0 + blk * EC, EC)])
    for blk in range(BLK):
        pltpu.sync_copy(part_hbm.at[1, s, pl.ds(blk * EC, EC)], r1)
        pltpu.async_copy(r1, ysp.at[idx2.at[blk]], s0, add=True)
        pltpu.make_async_copy(part_hbm.at[0, 0, pl.ds(0, EC)], r1, s0).wait()
    _zero_buf(r0)
    _zero_accum(accum, r0, row0)
    pltpu.sync_copy(src_hbm.at[wid], src_v)
    pltpu.sync_copy(dst_hbm.at[wid], dst_v)
    plsc.subcore_barrier()
    _ring_loop(ysp, accum, src_v, dst_v, [r0, r1, r2], [g0, g1, g2],
               [s0, s1, s2])
    plsc.subcore_barrier()
    pltpu.sync_copy(accum.at[pl.ds(row0, ROWS_PER_TILE)], out_hbm.at[c, s])


def _finish_body(p_ref, b_ref, o_ref):
    logits = p_ref[0, :N] + p_ref[1, :N] + b_ref[...]
    m = jnp.max(logits, axis=1, keepdims=True)
    lse = m + jnp.log(jnp.sum(jnp.exp(logits - m), axis=1, keepdims=True))
    o_ref[...] = logits - lse


def _finish(parts, b):
    return pl.pallas_call(
        _finish_body,
        out_shape=jax.ShapeDtypeStruct((N, D_OUT), jnp.float32),
    )(parts, b)


def kernel(x, edge_index, edge_mask, vertex_cnt, rule_cnt, W, b):
    del vertex_cnt, rule_cnt
    src = edge_index[0].astype(jnp.int32)
    dst = edge_index[1].astype(jnp.int32)
    msk = edge_mask.astype(jnp.int32)
    pad = EPAD - E
    src_f = jnp.pad(src, (0, pad)).reshape(NW, NBE)
    dst_f = jnp.pad(dst, (0, pad)).reshape(EPAD // EC, EC)
    msk_f = jnp.pad(msk, (0, pad)).reshape(EPAD // EC, EC)
    x_pad = jnp.pad(x, ((0, NPAD - N), (0, 0)))

    y, dst_m = _prep(x_pad, W, dst_f, msk_f)
    dst_m = dst_m.reshape(NW, NB, EC)

    p1 = _propagate1(y, src_f, dst_m)
    p2 = _propagate2(p1, src_f, dst_m)
    return _finish(p2.reshape(NC, NPAD, D_OUT), b.reshape(1, D_OUT))


# speedup vs baseline: 1.4847x; 1.4847x over previous
import functools

import jax
import jax.numpy as jnp
from jax import lax
from jax.experimental import pallas as pl
from jax.experimental.pallas import tpu as pltpu
from jax.experimental.pallas import tpu_sc as plsc

N = 10000
D_IN = 128
D_OUT = 64
E = 320000

NC = 2
NS = 16
NW = NC * NS

EC = 128
NB = 80
NBE = NB * EC
EPAD = NW * NBE
NPAD = 10240
ROWS_PER_TILE = NPAD // NS
BLK = ROWS_PER_TILE // EC
NDUMMY = NPAD - N
NBUF = 3
VL = 16


def _prep_body(x_ref, w_ref, dst_ref, msk_ref, y_ref, dstm_ref):
    y_ref[...] = lax.dot_general(
        x_ref[...], w_ref[...],
        (((1,), (1,)), ((), ())),
        preferred_element_type=jnp.float32,
    )
    r = lax.broadcasted_iota(jnp.int32, (EPAD // EC, EC), 0)
    cc = lax.broadcasted_iota(jnp.int32, (EPAD // EC, EC), 1)
    dummy = N + lax.rem(r * EC + cc, NDUMMY)
    dstm_ref[...] = jnp.where(msk_ref[...] != 0, dst_ref[...], dummy)


def _prep(x_pad, w, dst_f, msk_f):
    return pl.pallas_call(
        _prep_body,
        out_shape=(
            jax.ShapeDtypeStruct((NPAD, D_OUT), jnp.float32),
            jax.ShapeDtypeStruct((EPAD // EC, EC), jnp.int32),
        ),
    )(x_pad, w, dst_f, msk_f)


_MESH = plsc.VectorSubcoreMesh(
    core_axis_name="c", subcore_axis_name="s", num_cores=NC, num_subcores=NS
)
_CPARAMS = pltpu.CompilerParams(use_tc_tiling_on_sc=False)

_SC_SCRATCH = [
    pltpu.VMEM((NBE,), jnp.int32),
    pltpu.VMEM((NB, EC), jnp.int32),
    pltpu.VMEM((EC, D_OUT), jnp.float32),
    pltpu.VMEM((EC, D_OUT), jnp.float32),
    pltpu.VMEM((EC, D_OUT), jnp.float32),
    pltpu.VMEM((BLK, EC), jnp.int32),
    pltpu.VMEM_SHARED((NPAD, D_OUT), jnp.float32),
    pltpu.VMEM_SHARED((NPAD, D_OUT), jnp.float32),
    pltpu.SemaphoreType.DMA,
    pltpu.SemaphoreType.DMA,
    pltpu.SemaphoreType.DMA,
    pltpu.SemaphoreType.DMA,
    pltpu.SemaphoreType.DMA,
    pltpu.SemaphoreType.DMA,
]


def _zero_buf(buf):
    z = jnp.zeros((VL,), jnp.float32)

    def zrow(i, carry):
        for f in range(D_OUT // VL):
            buf[i, pl.ds(f * VL, VL)] = z
        return carry

    lax.fori_loop(0, EC, zrow, jnp.int32(0))


def _zero_accum(accum, r0, row0):
    for blk in range(BLK):
        pltpu.sync_copy(r0, accum.at[pl.ds(row0 + blk * EC, EC)])


def _fill_rowidx(idx2, row0):
    lanes = lax.broadcasted_iota(jnp.int32, (VL,), 0)
    for blk in range(BLK):
        for k in range(EC // VL):
            idx2[blk, pl.ds(k * VL, VL)] = row0 + blk * EC + k * VL + lanes


def _ring_loop(ysp, accum, src_v, dst_v, rows, gsems, ssems):
    def issue_gather(j, b):
        pltpu.async_copy(ysp.at[src_v.at[pl.ds(j * EC, EC)]], rows[b],
                         gsems[b])

    def wait(sem):
        pltpu.make_async_copy(ysp.at[pl.ds(0, EC)], rows[0], sem).wait()

    for b in range(NBUF):
        issue_gather(b, b)

    def gbody(gi, carry):
        g = gi * NBUF
        for b in range(NBUF):
            j = g + b
            bprev = (b - 1) % NBUF
            @pl.when((j >= 1) & (j - 1 + NBUF < NB))
            def _(j=j, bprev=bprev):
                wait(ssems[bprev])
                issue_gather(j - 1 + NBUF, bprev)

            @pl.when(j < NB)
            def _(j=j, b=b):
                wait(gsems[b])
                pltpu.async_copy(rows[b], accum.at[dst_v.at[j]], ssems[b],
                                 add=True)
        return carry

    lax.fori_loop(0, (NB + NBUF - 1) // NBUF, gbody, jnp.int32(0))
    for b in range(NBUF):
        wait(ssems[b])


@functools.partial(
    pl.kernel,
    out_type=jax.ShapeDtypeStruct((NC, NS, ROWS_PER_TILE, D_OUT), jnp.float32),
    mesh=_MESH,
    scratch_types=_SC_SCRATCH,
    compiler_params=_CPARAMS,
)
def _propagate1(y_hbm, src_hbm, dst_hbm, out_hbm,
                src_v, dst_v, r0, r1, r2, idx2, ysp, accum,
                g0, g1, g2, s0, s1, s2):
    c = lax.axis_index("c")
    s = lax.axis_index("s")
    wid = c * NS + s
    row0 = s * ROWS_PER_TILE
    pltpu.async_copy(y_hbm.at[pl.ds(row0, ROWS_PER_TILE)],
                     ysp.at[pl.ds(row0, ROWS_PER_TILE)], g0)
    pltpu.async_copy(src_hbm.at[wid], src_v, g1)
    pltpu.async_copy(dst_hbm.at[wid], dst_v, g2)
    _zero_buf(r0)
    _zero_accum(accum, r0, row0)
    pltpu.make_async_copy(y_hbm.at[pl.ds(row0, ROWS_PER_TILE)],
                          ysp.at[pl.ds(row0, ROWS_PER_TILE)], g0).wait()
    pltpu.make_async_copy(src_hbm.at[wid], src_v, g1).wait()
    pltpu.make_async_copy(dst_hbm.at[wid], dst_v, g2).wait()
    plsc.subcore_barrier()
    _ring_loop(ysp, accum, src_v, dst_v, [r0, r1, r2], [g0, g1, g2],
               [s0, s1, s2])
    plsc.subcore_barrier()
    pltpu.sync_copy(accum.at[pl.ds(row0, ROWS_PER_TILE)], out_hbm.at[c, s])


@functools.partial(
    pl.kernel,
    out_type=jax.ShapeDtypeStruct((NC, NS, ROWS_PER_TILE, D_OUT), jnp.float32),
    mesh=_MESH,
    scratch_types=_SC_SCRATCH,
    compiler_params=_CPARAMS,
)
def _propagate2(part_hbm, src_hbm, dst_hbm, out_hbm,
                src_v, dst_v, r0, r1, r2, idx2, ysp, accum,
                g0, g1, g2, s0, s1, s2):
    c = lax.axis_index("c")
    s = lax.axis_index("s")
    wid = c * NS + s
    row0 = s * ROWS_PER_TILE
    pltpu.async_copy(src_hbm.at[wid], src_v, g1)
    pltpu.async_copy(dst_hbm.at[wid], dst_v, g2)
    _fill_rowidx(idx2, row0)
    for blk in range(BLK):
        pltpu.sync_copy(part_hbm.at[0, s, pl.ds(blk * EC, EC)], r0)
        pltpu.sync_copy(r0, ysp.at[pl.ds(row0 + blk * EC, EC)])
    for blk in range(BLK):
        pltpu.sync_copy(part_hbm.at[1, s, pl.ds(blk * EC, EC)], r1)
        pltpu.async_copy(r1, ysp.at[idx2.at[blk]], s0, add=True)
        pltpu.make_async_copy(part_hbm.at[0, 0, pl.ds(0, EC)], r1, s0).wait()
    _zero_buf(r0)
    _zero_accum(accum, r0, row0)
    pltpu.make_async_copy(src_hbm.at[wid], src_v, g1).wait()
    pltpu.make_async_copy(dst_hbm.at[wid], dst_v, g2).wait()
    plsc.subcore_barrier()
    _ring_loop(ysp, accum, src_v, dst_v, [r0, r1, r2], [g0, g1, g2],
               [s0, s1, s2])
    plsc.subcore_barrier()
    pltpu.sync_copy(accum.at[pl.ds(row0, ROWS_PER_TILE)], out_hbm.at[c, s])


def _finish_body(p_ref, b_ref, o_ref):
    logits = p_ref[0, :N] + p_ref[1, :N] + b_ref[...]
    m = jnp.max(logits, axis=1, keepdims=True)
    lse = m + jnp.log(jnp.sum(jnp.exp(logits - m), axis=1, keepdims=True))
    o_ref[...] = logits - lse


def _finish(parts, b):
    return pl.pallas_call(
        _finish_body,
        out_shape=jax.ShapeDtypeStruct((N, D_OUT), jnp.float32),
    )(parts, b)


def kernel(x, edge_index, edge_mask, vertex_cnt, rule_cnt, W, b):
    del vertex_cnt, rule_cnt
    src = edge_index[0].astype(jnp.int32)
    dst = edge_index[1].astype(jnp.int32)
    msk = edge_mask.astype(jnp.int32)
    pad = EPAD - E
    src_f = jnp.pad(src, (0, pad)).reshape(NW, NBE)
    dst_f = jnp.pad(dst, (0, pad)).reshape(EPAD // EC, EC)
    msk_f = jnp.pad(msk, (0, pad)).reshape(EPAD // EC, EC)
    x_pad = jnp.pad(x, ((0, NPAD - N), (0, 0)))

    y, dst_m = _prep(x_pad, W, dst_f, msk_f)
    dst_m = dst_m.reshape(NW, NB, EC)

    p1 = _propagate1(y, src_f, dst_m)
    p2 = _propagate2(p1, src_f, dst_m)
    return _finish(p2.reshape(NC, NPAD, D_OUT), b.reshape(1, D_OUT))
